# baseline (speedup 1.0000x reference)
import functools

import jax
import jax.numpy as jnp
from jax import lax
from jax.experimental import pallas as pl
from jax.experimental.pallas import tpu as pltpu
from jax.experimental.pallas import tpu_sc as plsc

N_NODES = 10000
N_EDGES = 320000
DIM = 128
ALPHA = 0.2

NC, NS, L = 2, 16, 16
NW = NC * NS
CPT = DIM // NW
EB = 4000
NB = N_EDGES // EB
NBP = NB // 2
GPB = EB // L
DRNG = 320


def _sc_agg_body(ht_hbm, s1_hbm, s2_hbm, src_hbm, dst_hbm, num_out, den_out,
                 s1_v, s2_v, hcol_v, acc_v, den_v,
                 src_a, dst_a, src_b, dst_b, sem_a, sem_b):
  cid = lax.axis_index("c")
  sid = lax.axis_index("s")
  wid = sid * NC + cid
  zeros16 = jnp.zeros((L,), jnp.float32)

  pltpu.sync_copy(s1_hbm, s1_v)
  pltpu.sync_copy(s2_hbm, s2_v)
  pltpu.sync_copy(ht_hbm.at[wid], hcol_v)

  def _zero_acc(i, carry):
    for c in range(CPT):
      acc_v[c, pl.ds(L * i, L)] = zeros16
    return carry
  lax.fori_loop(0, N_NODES // L, _zero_acc, 0)

  def _zero_den(i, carry):
    den_v[pl.ds(L * i, L)] = zeros16
    return carry
  lax.fori_loop(0, DRNG // L, _zero_den, 0)

  dlo = wid * DRNG
  cvecs = [jnp.full((L,), c, jnp.int32) for c in range(CPT)]

  def _process(src_blk, dst_blk):
    @plsc.parallel_loop(0, GPB, unroll=16)
    def _group(j):
      sv = src_blk[pl.ds(L * j, L)]
      dv = dst_blk[pl.ds(L * j, L)]
      e = plsc.load_gather(s1_v, [sv]) + plsc.load_gather(s2_v, [dv])
      e = jnp.where(e > 0, e, ALPHA * e)
      p = jnp.exp(e)
      for c in range(CPT):
        hv = plsc.load_gather(hcol_v, [cvecs[c], sv])
        plsc.addupdate_scatter(acc_v, [cvecs[c], dv], hv * p)
      dloc = dv - dlo
      m = (dloc >= 0) & (dloc < DRNG)
      plsc.addupdate_scatter(den_v, [jnp.clip(dloc, 0, DRNG - 1)], p, mask=m)

  def _fire(b, sblk, dblk, sem):
    pltpu.async_copy(src_hbm.at[pl.ds(b * EB, EB)], sblk, sem)
    pltpu.async_copy(dst_hbm.at[pl.ds(b * EB, EB)], dblk, sem)

  def _wait(sblk, dblk, sem):
    pltpu.make_async_copy(src_hbm.at[pl.ds(0, EB)], sblk, sem).wait()
    pltpu.make_async_copy(dst_hbm.at[pl.ds(0, EB)], dblk, sem).wait()

  _fire(0, src_a, dst_a, sem_a)

  def _bpair(t, carry):
    _fire(2 * t + 1, src_b, dst_b, sem_b)
    _wait(src_a, dst_a, sem_a)
    _process(src_a, dst_a)

    @pl.when(t < NBP - 1)
    def _prefetch():
      _fire(2 * t + 2, src_a, dst_a, sem_a)

    _wait(src_b, dst_b, sem_b)
    _process(src_b, dst_b)
    return carry

  lax.fori_loop(0, NBP, _bpair, 0)

  pltpu.sync_copy(acc_v, num_out.at[wid])
  pltpu.sync_copy(den_v, den_out.at[pl.ds(dlo, DRNG)])


_sc_agg = functools.partial(
    pl.kernel,
    out_type=[
        jax.ShapeDtypeStruct((NW, CPT, N_NODES), jnp.float32),
        jax.ShapeDtypeStruct((NW * DRNG,), jnp.float32),
    ],
    mesh=plsc.VectorSubcoreMesh(core_axis_name="c", subcore_axis_name="s"),
    scratch_types=[
        pltpu.VMEM((N_NODES,), jnp.float32),
        pltpu.VMEM((N_NODES,), jnp.float32),
        pltpu.VMEM((CPT, N_NODES), jnp.float32),
        pltpu.VMEM((CPT, N_NODES), jnp.float32),
        pltpu.VMEM((DRNG,), jnp.float32),
        pltpu.VMEM((EB,), jnp.int32),
        pltpu.VMEM((EB,), jnp.int32),
        pltpu.VMEM((EB,), jnp.int32),
        pltpu.VMEM((EB,), jnp.int32),
        pltpu.SemaphoreType.DMA,
        pltpu.SemaphoreType.DMA,
    ],
    compiler_params=pltpu.CompilerParams(needs_layout_passes=False),
)(_sc_agg_body)


def _tc_prep_body(x_ref, w_ref, a8_ref, ht_ref, s_ref):
  ht = lax.dot_general(w_ref[...], x_ref[...], (((1,), (1,)), ((), ())),
                       preferred_element_type=jnp.float32)
  ht_ref[...] = ht
  s_ref[...] = lax.dot_general(a8_ref[...], ht, (((1,), (0,)), ((), ())),
                               preferred_element_type=jnp.float32)


_tc_prep = pl.pallas_call(
    _tc_prep_body,
    out_shape=[
        jax.ShapeDtypeStruct((DIM, N_NODES), jnp.float32),
        jax.ShapeDtypeStruct((8, N_NODES), jnp.float32),
    ],
)


def _tc_fin_body(numt_ref, den_ref, out_ref):
  att = numt_ref[...] / (den_ref[...] + 1e-10)
  att = jnp.where(att > 0, att, jnp.exp(att) - 1.0)
  out_ref[...] = att.T


_tc_fin = pl.pallas_call(
    _tc_fin_body,
    out_shape=jax.ShapeDtypeStruct((N_NODES, DIM), jnp.float32),
)


@jax.jit
def kernel(node_features, edge_index, W, a_w):
  a8 = jnp.zeros((8, DIM), jnp.float32)
  a8 = a8.at[0].set(a_w[0, :DIM]).at[1].set(a_w[0, DIM:])
  ht, s8 = _tc_prep(node_features, W, a8)
  ht4 = ht.reshape(NW, CPT, N_NODES)
  numt, den = _sc_agg(ht4, s8[0], s8[1], edge_index[0], edge_index[1])
  num128 = numt.reshape(DIM, N_NODES)
  den_row = den[:N_NODES].reshape(1, N_NODES)
  return _tc_fin(num128, den_row)


# speedup vs baseline: 1.3708x; 1.3708x over previous
import functools

import jax
import jax.numpy as jnp
from jax import lax
from jax.experimental import pallas as pl
from jax.experimental.pallas import tpu as pltpu
from jax.experimental.pallas import tpu_sc as plsc

N_NODES = 10000
N_EDGES = 320000
DIM = 128
ALPHA = 0.2

NC, NS, L = 2, 16, 16
NW = NC * NS
CPT = DIM // NW
EB = 6400
NB = N_EDGES // EB
NBP = NB // 2
GPB = EB // L
DRNG = 320


def _sc_agg_body(ht_hbm, s1_hbm, s2_hbm, src_hbm, dst_hbm, num_out, den_out,
                 s1_v, s2_v, hcol_v, acc_v, den_v,
                 src_a, dst_a, src_b, dst_b, sem_a, sem_b):
  cid = lax.axis_index("c")
  sid = lax.axis_index("s")
  wid = sid * NC + cid
  zeros16 = jnp.zeros((L,), jnp.float32)

  pltpu.sync_copy(s1_hbm, s1_v)
  pltpu.sync_copy(s2_hbm, s2_v)
  pltpu.sync_copy(ht_hbm.at[wid], hcol_v)

  def _zero_acc(i, carry):
    for c in range(CPT):
      acc_v[c, pl.ds(L * i, L)] = zeros16
    return carry
  lax.fori_loop(0, N_NODES // L, _zero_acc, 0)

  def _zero_den(i, carry):
    den_v[pl.ds(L * i, L)] = zeros16
    return carry
  lax.fori_loop(0, DRNG // L, _zero_den, 0)

  dlo = wid * DRNG
  cvecs = [jnp.full((L,), c, jnp.int32) for c in range(CPT)]

  def _process(src_blk, dst_blk):
    @plsc.parallel_loop(0, GPB, unroll=8)
    def _group(j):
      sv = src_blk[pl.ds(L * j, L)]
      dv = dst_blk[pl.ds(L * j, L)]
      e = plsc.load_gather(s1_v, [sv]) + plsc.load_gather(s2_v, [dv])
      e = jnp.where(e > 0, e, ALPHA * e)
      p = jnp.exp(e)
      for c in range(CPT):
        hv = plsc.load_gather(hcol_v, [cvecs[c], sv])
        plsc.addupdate_scatter(acc_v, [cvecs[c], dv], hv * p)
      dloc = dv - dlo
      m = (dloc >= 0) & (dloc < DRNG)
      plsc.addupdate_scatter(den_v, [jnp.clip(dloc, 0, DRNG - 1)], p, mask=m)

  def _fire(b, sblk, dblk, sem):
    pltpu.async_copy(src_hbm.at[pl.ds(b * EB, EB)], sblk, sem)
    pltpu.async_copy(dst_hbm.at[pl.ds(b * EB, EB)], dblk, sem)

  def _wait(sblk, dblk, sem):
    pltpu.make_async_copy(src_hbm.at[pl.ds(0, EB)], sblk, sem).wait()
    pltpu.make_async_copy(dst_hbm.at[pl.ds(0, EB)], dblk, sem).wait()

  _fire(0, src_a, dst_a, sem_a)

  def _bpair(t, carry):
    _fire(2 * t + 1, src_b, dst_b, sem_b)
    _wait(src_a, dst_a, sem_a)
    _process(src_a, dst_a)

    @pl.when(t < NBP - 1)
    def _prefetch():
      _fire(2 * t + 2, src_a, dst_a, sem_a)

    _wait(src_b, dst_b, sem_b)
    _process(src_b, dst_b)
    return carry

  lax.fori_loop(0, NBP, _bpair, 0)

  pltpu.sync_copy(acc_v, num_out.at[wid])
  pltpu.sync_copy(den_v, den_out.at[pl.ds(dlo, DRNG)])


_sc_agg = functools.partial(
    pl.kernel,
    out_type=[
        jax.ShapeDtypeStruct((NW, CPT, N_NODES), jnp.float32),
        jax.ShapeDtypeStruct((NW * DRNG,), jnp.float32),
    ],
    mesh=plsc.VectorSubcoreMesh(core_axis_name="c", subcore_axis_name="s"),
    scratch_types=[
        pltpu.VMEM((N_NODES,), jnp.float32),
        pltpu.VMEM((N_NODES,), jnp.float32),
        pltpu.VMEM((CPT, N_NODES), jnp.float32),
        pltpu.VMEM((CPT, N_NODES), jnp.float32),
        pltpu.VMEM((DRNG,), jnp.float32),
        pltpu.VMEM((EB,), jnp.int32),
        pltpu.VMEM((EB,), jnp.int32),
        pltpu.VMEM((EB,), jnp.int32),
        pltpu.VMEM((EB,), jnp.int32),
        pltpu.SemaphoreType.DMA,
        pltpu.SemaphoreType.DMA,
    ],
    compiler_params=pltpu.CompilerParams(needs_layout_passes=False),
)(_sc_agg_body)


def _tc_prep_body(x_ref, w_ref, a8_ref, ht_ref, s_ref):
  ht = lax.dot_general(w_ref[...], x_ref[...], (((1,), (1,)), ((), ())),
                       preferred_element_type=jnp.float32)
  ht_ref[...] = ht
  s_ref[...] = lax.dot_general(a8_ref[...], ht, (((1,), (0,)), ((), ())),
                               preferred_element_type=jnp.float32)


_tc_prep = pl.pallas_call(
    _tc_prep_body,
    out_shape=[
        jax.ShapeDtypeStruct((DIM, N_NODES), jnp.float32),
        jax.ShapeDtypeStruct((8, N_NODES), jnp.float32),
    ],
)


def _tc_fin_body(numt_ref, den_ref, out_ref):
  att = numt_ref[...] / (den_ref[...] + 1e-10)
  att = jnp.where(att > 0, att, jnp.exp(att) - 1.0)
  out_ref[...] = att.T


_tc_fin = pl.pallas_call(
    _tc_fin_body,
    out_shape=jax.ShapeDtypeStruct((N_NODES, DIM), jnp.float32),
)


@jax.jit
def kernel(node_features, edge_index, W, a_w):
  a8 = jnp.zeros((8, DIM), jnp.float32)
  a8 = a8.at[0].set(a_w[0, :DIM]).at[1].set(a_w[0, DIM:])
  ht, s8 = _tc_prep(node_features, W, a8)
  ht4 = ht.reshape(NW, CPT, N_NODES)
  numt, den = _sc_agg(ht4, s8[0], s8[1], edge_index[0], edge_index[1])
  num128 = numt.reshape(DIM, N_NODES)
  den_row = den[:N_NODES].reshape(1, N_NODES)
  return _tc_fin(num128, den_row)


# speedup vs baseline: 1.3883x; 1.0128x over previous
import functools

import jax
import jax.numpy as jnp
from jax import lax
from jax.experimental import pallas as pl
from jax.experimental.pallas import tpu as pltpu
from jax.experimental.pallas import tpu_sc as plsc

N_NODES = 10000
N_EDGES = 320000
DIM = 128
ALPHA = 0.2

NC, NS, L = 2, 16, 16
NW = NC * NS
CPT = DIM // NW
EB = 6400
NB = N_EDGES // EB
NBP = NB // 2
GPB = EB // L
DRNG = 320


def _sc_agg_body(ht_hbm, s1_hbm, s2_hbm, src_hbm, dst_hbm, num_out, den_out,
                 s1_v, s2_v, hcol_v, acc_v, den_v,
                 src_a, dst_a, src_b, dst_b, sem_a, sem_b):
  cid = lax.axis_index("c")
  sid = lax.axis_index("s")
  wid = sid * NC + cid
  zeros16 = jnp.zeros((L,), jnp.float32)

  pltpu.sync_copy(s1_hbm, s1_v)
  pltpu.sync_copy(s2_hbm, s2_v)
  pltpu.sync_copy(ht_hbm.at[wid], hcol_v)

  def _zero_acc(i, carry):
    for c in range(CPT):
      acc_v[c, pl.ds(L * i, L)] = zeros16
    return carry
  lax.fori_loop(0, N_NODES // L, _zero_acc, 0)

  def _zero_den(i, carry):
    den_v[pl.ds(L * i, L)] = zeros16
    return carry
  lax.fori_loop(0, DRNG // L, _zero_den, 0)

  dlo = wid * DRNG
  cvecs = [jnp.full((L,), c, jnp.int32) for c in range(CPT)]

  def _process(src_blk, dst_blk):
    @plsc.parallel_loop(0, GPB, unroll=4)
    def _group(j):
      sv = src_blk[pl.ds(L * j, L)]
      dv = dst_blk[pl.ds(L * j, L)]
      e = plsc.load_gather(s1_v, [sv]) + plsc.load_gather(s2_v, [dv])
      e = jnp.where(e > 0, e, ALPHA * e)
      p = jnp.exp(e)
      for c in range(CPT):
        hv = plsc.load_gather(hcol_v, [cvecs[c], sv])
        plsc.addupdate_scatter(acc_v, [cvecs[c], dv], hv * p)
      dloc = dv - dlo
      m = (dloc >= 0) & (dloc < DRNG)
      plsc.addupdate_scatter(den_v, [jnp.clip(dloc, 0, DRNG - 1)], p, mask=m)

  def _fire(b, sblk, dblk, sem):
    pltpu.async_copy(src_hbm.at[pl.ds(b * EB, EB)], sblk, sem)
    pltpu.async_copy(dst_hbm.at[pl.ds(b * EB, EB)], dblk, sem)

  def _wait(sblk, dblk, sem):
    pltpu.make_async_copy(src_hbm.at[pl.ds(0, EB)], sblk, sem).wait()
    pltpu.make_async_copy(dst_hbm.at[pl.ds(0, EB)], dblk, sem).wait()

  _fire(0, src_a, dst_a, sem_a)

  def _bpair(t, carry):
    _fire(2 * t + 1, src_b, dst_b, sem_b)
    _wait(src_a, dst_a, sem_a)
    _process(src_a, dst_a)

    @pl.when(t < NBP - 1)
    def _prefetch():
      _fire(2 * t + 2, src_a, dst_a, sem_a)

    _wait(src_b, dst_b, sem_b)
    _process(src_b, dst_b)
    return carry

  lax.fori_loop(0, NBP, _bpair, 0)

  pltpu.sync_copy(acc_v, num_out.at[wid])
  pltpu.sync_copy(den_v, den_out.at[pl.ds(dlo, DRNG)])


_sc_agg = functools.partial(
    pl.kernel,
    out_type=[
        jax.ShapeDtypeStruct((NW, CPT, N_NODES), jnp.float32),
        jax.ShapeDtypeStruct((NW * DRNG,), jnp.float32),
    ],
    mesh=plsc.VectorSubcoreMesh(core_axis_name="c", subcore_axis_name="s"),
    scratch_types=[
        pltpu.VMEM((N_NODES,), jnp.float32),
        pltpu.VMEM((N_NODES,), jnp.float32),
        pltpu.VMEM((CPT, N_NODES), jnp.float32),
        pltpu.VMEM((CPT, N_NODES), jnp.float32),
        pltpu.VMEM((DRNG,), jnp.float32),
        pltpu.VMEM((EB,), jnp.int32),
        pltpu.VMEM((EB,), jnp.int32),
        pltpu.VMEM((EB,), jnp.int32),
        pltpu.VMEM((EB,), jnp.int32),
        pltpu.SemaphoreType.DMA,
        pltpu.SemaphoreType.DMA,
    ],
    compiler_params=pltpu.CompilerParams(needs_layout_passes=False),
)(_sc_agg_body)


def _tc_prep_body(x_ref, w_ref, a8_ref, ht_ref, s_ref):
  ht = lax.dot_general(w_ref[...], x_ref[...], (((1,), (1,)), ((), ())),
                       preferred_element_type=jnp.float32)
  ht_ref[...] = ht
  s_ref[...] = lax.dot_general(a8_ref[...], ht, (((1,), (0,)), ((), ())),
                               preferred_element_type=jnp.float32)


_tc_prep = pl.pallas_call(
    _tc_prep_body,
    out_shape=[
        jax.ShapeDtypeStruct((DIM, N_NODES), jnp.float32),
        jax.ShapeDtypeStruct((8, N_NODES), jnp.float32),
    ],
)


def _tc_fin_body(numt_ref, den_ref, out_ref):
  att = numt_ref[...] / (den_ref[...] + 1e-10)
  att = jnp.where(att > 0, att, jnp.exp(att) - 1.0)
  out_ref[...] = att.T


_tc_fin = pl.pallas_call(
    _tc_fin_body,
    out_shape=jax.ShapeDtypeStruct((N_NODES, DIM), jnp.float32),
)


@jax.jit
def kernel(node_features, edge_index, W, a_w):
  a8 = jnp.zeros((8, DIM), jnp.float32)
  a8 = a8.at[0].set(a_w[0, :DIM]).at[1].set(a_w[0, DIM:])
  ht, s8 = _tc_prep(node_features, W, a8)
  ht4 = ht.reshape(NW, CPT, N_NODES)
  numt, den = _sc_agg(ht4, s8[0], s8[1], edge_index[0], edge_index[1])
  num128 = numt.reshape(DIM, N_NODES)
  den_row = den[:N_NODES].reshape(1, N_NODES)
  return _tc_fin(num128, den_row)
